# baseline (speedup 1.0000x reference)
import functools

import jax
import jax.numpy as jnp
from jax import lax
from jax.experimental import pallas as pl
from jax.experimental.pallas import tpu as pltpu
from jax.experimental.pallas import tpu_sc as plsc

N = 50000
D = 128
NUM_SCALES = 6
NUM_LAYERS = 4
NK = 2 * NUM_SCALES + 2

RB = 1024
RB2 = 2048
NP = 51200

EPS = 1e-5


def _dot_t(x, w):
    return lax.dot_general(x.astype(jnp.bfloat16), w.astype(jnp.bfloat16),
                           (((1,), (1,)), ((), ())),
                           preferred_element_type=jnp.float32)


def _msg_body(feat_ref, w_ref, msg_ref, temp0_ref):
    x = feat_ref[...]
    for k in range(NK):
        msg_ref[k] = _dot_t(x, w_ref[k])
    temp0_ref[...] = _dot_t(x, w_ref[NK])


def _msg_call(feat_p, w_all):
    grid = (NP // RB,)
    return pl.pallas_call(
        _msg_body,
        grid=grid,
        in_specs=[
            pl.BlockSpec((RB, D), lambda i: (i, 0)),
            pl.BlockSpec((NK + 1, D, D), lambda i: (0, 0, 0)),
        ],
        out_specs=[
            pl.BlockSpec((NK, RB, D), lambda i: (0, i, 0)),
            pl.BlockSpec((RB, D), lambda i: (i, 0)),
        ],
        out_shape=[
            jax.ShapeDtypeStruct((NK, NP, D), jnp.float32),
            jax.ShapeDtypeStruct((NP, D), jnp.float32),
        ],
    )(feat_p, w_all)


def _gn(x, gamma, beta):
    mean = jnp.mean(x, axis=-1, keepdims=True)
    var = jnp.mean((x - mean) ** 2, axis=-1, keepdims=True)
    xh = (x - mean) * lax.rsqrt(var + EPS)
    return xh * gamma + beta


def _post_body(temp_ref, res_ref, g1_ref, b1_ref, w2_ref, g2_ref, b2_ref,
               out_ref):
    t = temp_ref[...]
    x = _gn(t, g1_ref[...], b1_ref[...])
    x = jnp.maximum(x, 0.0)
    x = lax.dot_general(x, w2_ref[...], (((1,), (1,)), ((), ())),
                        preferred_element_type=jnp.float32)
    x = _gn(x, g2_ref[...], b2_ref[...])
    out_ref[...] = jnp.maximum(x + res_ref[...], 0.0)


def _post_call(temp_p, res_p, g1, b1, w2, g2, b2):
    grid = (NP // RB2,)
    return pl.pallas_call(
        _post_body,
        grid=grid,
        in_specs=[
            pl.BlockSpec((RB2, D), lambda i: (i, 0)),
            pl.BlockSpec((RB2, D), lambda i: (i, 0)),
            pl.BlockSpec((1, D), lambda i: (0, 0)),
            pl.BlockSpec((1, D), lambda i: (0, 0)),
            pl.BlockSpec((D, D), lambda i: (0, 0)),
            pl.BlockSpec((1, D), lambda i: (0, 0)),
            pl.BlockSpec((1, D), lambda i: (0, 0)),
        ],
        out_specs=pl.BlockSpec((RB2, D), lambda i: (i, 0)),
        out_shape=jax.ShapeDtypeStruct((NP, D), jnp.float32),
    )(temp_p, res_p, g1, b1, w2, g2, b2)


def _fused_body(temp_ref, res_ref, g1_ref, b1_ref, w2_ref, g2_ref, b2_ref,
                w_ref, feat_ref, msg_ref, temp0_ref):
    x = _gn(temp_ref[...], g1_ref[...], b1_ref[...])
    x = jnp.maximum(x, 0.0)
    x = _dot_t(x, w2_ref[...])
    x = _gn(x, g2_ref[...], b2_ref[...])
    f = jnp.maximum(x + res_ref[...], 0.0)
    feat_ref[...] = f
    for k in range(NK):
        msg_ref[k] = _dot_t(f, w_ref[k])
    temp0_ref[...] = _dot_t(f, w_ref[NK])


def _fused_call(temp_p, res_p, g1, b1, w2, g2, b2, w_all):
    grid = (NP // RB,)
    return pl.pallas_call(
        _fused_body,
        grid=grid,
        in_specs=[
            pl.BlockSpec((RB, D), lambda i: (i, 0)),
            pl.BlockSpec((RB, D), lambda i: (i, 0)),
            pl.BlockSpec((1, D), lambda i: (0, 0)),
            pl.BlockSpec((1, D), lambda i: (0, 0)),
            pl.BlockSpec((D, D), lambda i: (0, 0)),
            pl.BlockSpec((1, D), lambda i: (0, 0)),
            pl.BlockSpec((1, D), lambda i: (0, 0)),
            pl.BlockSpec((NK + 1, D, D), lambda i: (0, 0, 0)),
        ],
        out_specs=[
            pl.BlockSpec((RB, D), lambda i: (i, 0)),
            pl.BlockSpec((NK, RB, D), lambda i: (0, i, 0)),
            pl.BlockSpec((RB, D), lambda i: (i, 0)),
        ],
        out_shape=[
            jax.ShapeDtypeStruct((NP, D), jnp.float32),
            jax.ShapeDtypeStruct((NK, NP, D), jnp.float32),
            jax.ShapeDtypeStruct((NP, D), jnp.float32),
        ],
    )(temp_p, res_p, g1, b1, w2, g2, b2, w_all)


NCH = 10
CH = 5120
CHT = CH // 16
WIN = 128
GRP = 16 * WIN
E_TOT = 12 * 50000 + 2 * 5000
E_CAP = E_TOT + NCH * GRP

_sc_agg_cache = []


def _sc_agg(msg, temp0, gidx, sidx, meta):
    if not _sc_agg_cache:
        mesh = plsc.VectorSubcoreMesh(core_axis_name="c",
                                      subcore_axis_name="s")
        _sc_agg_cache.append(functools.partial(
            pl.kernel,
            mesh=mesh,
            out_type=jax.ShapeDtypeStruct((NP, D), jnp.float32),
            compiler_params=pltpu.CompilerParams(needs_layout_passes=False),
            scratch_types=[
                pltpu.VMEM((2, WIN), jnp.int32),
                pltpu.VMEM((2, WIN), jnp.int32),
                pltpu.VMEM((2, WIN, D), jnp.float32),
                pltpu.VMEM((2, 16), jnp.int32),
                pltpu.VMEM_SHARED((CH + 16, D), jnp.float32),
                pltpu.VMEM_SHARED((CH + 16, D), jnp.float32),
                pltpu.SemaphoreType.DMA,
                pltpu.SemaphoreType.DMA,
                pltpu.SemaphoreType.DMA,
                pltpu.SemaphoreType.DMA,
            ],
        )(_sc_agg_body))
    return _sc_agg_cache[0](msg, temp0, gidx, sidx, meta)


def _sc_agg_body(msg_hbm, temp0_hbm, gidx_hbm, sidx_hbm, meta_hbm, out_hbm,
                 gbuf, sbuf, rows, metav, acc0, acc1, sem, sem_i, sem_s,
                 sem_o):
    core = lax.axis_index("c")
    tile = lax.axis_index("s")
    pltpu.sync_copy(meta_hbm, metav)
    mv_off = metav[0]
    mv_nwt = metav[1]
    accs = (acc0, acc1)
    NPC = NCH // 2

    def sget(vec, j):
        return jnp.sum(jnp.where(lax.iota(jnp.int32, 16) == j, vec, 0))

    r0 = tile * CHT

    def seed_start(j, b):
        base = (core * NPC + j) * CH
        pltpu.async_copy(temp0_hbm.at[pl.ds(base + r0, CHT)],
                         accs[b].at[pl.ds(r0, CHT)], sem_s)

    def seed_wait(b):
        pltpu.make_async_copy(temp0_hbm.at[pl.ds(0, CHT)],
                              accs[b].at[pl.ds(r0, CHT)], sem_s).wait()

    def out_start(j, b):
        base = (core * NPC + j) * CH
        pltpu.async_copy(accs[b].at[pl.ds(r0, CHT)],
                         out_hbm.at[pl.ds(base + r0, CHT)], sem_o)

    def out_wait(b):
        pltpu.make_async_copy(accs[b].at[pl.ds(r0, CHT)],
                              out_hbm.at[pl.ds(0, CHT)], sem_o).wait()

    def windows(j, bb):
        acc = accs[bb]
        c = core * NPC + j
        off_c = sget(mv_off, c)
        nwt_c = sget(mv_nwt, c)

        def idx_start(k):
            b = k & 1
            e_off = pl.multiple_of(off_c + (k * 16 + tile) * WIN, WIN)
            pltpu.async_copy(gidx_hbm.at[pl.ds(e_off, WIN)], gbuf.at[b],
                             sem_i)
            pltpu.async_copy(sidx_hbm.at[pl.ds(e_off, WIN)], sbuf.at[b],
                             sem_i)

        def idx_wait(k):
            b = k & 1
            pltpu.make_async_copy(gidx_hbm.at[pl.ds(0, WIN)], gbuf.at[b],
                                  sem_i).wait()
            pltpu.make_async_copy(sidx_hbm.at[pl.ds(0, WIN)], sbuf.at[b],
                                  sem_i).wait()

        def gather_start(k):
            b = k & 1
            pltpu.async_copy(msg_hbm.at[gbuf.at[b]], rows.at[b], sem)

        def gather_wait(k):
            b = k & 1
            pltpu.make_async_copy(msg_hbm.at[gbuf.at[b]], rows.at[b],
                                  sem).wait()

        @pl.when(nwt_c > 0)
        def _():
            idx_start(0)

        @pl.when(nwt_c > 1)
        def _():
            idx_start(1)

        @pl.when(nwt_c > 0)
        def _():
            idx_wait(0)
            gather_start(0)

        def body(k, carry):
            @pl.when(k + 1 < nwt_c)
            def _():
                idx_wait(k + 1)
                gather_start(k + 1)

            gather_wait(k)
            b = k & 1
            pltpu.sync_copy(rows.at[b], acc.at[sbuf.at[b]], add=True)

            @pl.when(k + 2 < nwt_c)
            def _():
                idx_start(k + 2)

            return carry

        lax.fori_loop(0, nwt_c, body, 0)

    seed_start(0, 0)
    seed_wait(0)
    plsc.subcore_barrier()
    for j in range(NPC):
        b = j & 1
        if j == 0 and j + 1 < NPC:
            seed_start(j + 1, 1 - b)
        windows(j, b)
        plsc.subcore_barrier()
        out_start(j, b)
        if j + 1 < NPC:
            seed_wait(1 - b)
            plsc.subcore_barrier()
            if j + 2 < NPC:
                out_wait(b)
                seed_start(j + 2, b)
    out_wait(0)
    out_wait(1)


def _edge_prep(u_all, gidx_all):
    cid = u_all // CH
    ind = (cid[:, None] == jnp.arange(NCH, dtype=jnp.int32)[None, :])
    ranks = jnp.cumsum(ind.astype(jnp.int32), axis=0) - 1
    cnt = jnp.sum(ind.astype(jnp.int32), axis=0)
    pc = ((cnt + GRP - 1) // GRP) * GRP
    off = jnp.concatenate([jnp.zeros((1,), jnp.int32),
                           jnp.cumsum(pc)]).astype(jnp.int32)
    rank_e = jnp.sum(jnp.where(ind, ranks, 0), axis=1)
    pos = off[cid] + rank_e
    ar = jnp.arange(E_CAP, dtype=jnp.int32)
    dummy_g = ar % 1024
    dummy_s = CH + (ar % 16)
    gidx_pad = dummy_g.at[pos].add(gidx_all - (pos % 1024))
    sidx_pad = dummy_s.at[pos].add(u_all % CH - (CH + pos % 16))
    nwt = pc // GRP
    meta = jnp.zeros((2, 16), jnp.int32).at[0, :NCH].set(off[:NCH]).at[
        1, :NCH].set(nwt)
    return gidx_pad, sidx_pad, meta


def kernel(feat, pre_u, pre_v, suc_u, suc_v, left_u, left_v, right_u,
           right_v, W_ctr, W_edge, gamma1, beta1, W_ctr2, gamma2, beta2):
    u_all = jnp.concatenate([pre_u.reshape(-1), suc_u.reshape(-1),
                             left_u, right_u]).astype(jnp.int32)
    v_all = jnp.concatenate([pre_v.reshape(-1), suc_v.reshape(-1),
                             left_v, right_v]).astype(jnp.int32)
    s_all = jnp.concatenate([
        jnp.repeat(jnp.arange(NUM_SCALES, dtype=jnp.int32), pre_u.shape[1]),
        jnp.repeat(jnp.arange(NUM_SCALES, 2 * NUM_SCALES, dtype=jnp.int32),
                   suc_u.shape[1]),
        jnp.full(left_u.shape, 2 * NUM_SCALES, jnp.int32),
        jnp.full(right_u.shape, 2 * NUM_SCALES + 1, jnp.int32),
    ])
    gidx_all = s_all * NP + v_all
    gidx_pad, sidx_pad, meta = _edge_prep(u_all, gidx_all)

    feat_p = jnp.zeros((NP, D), jnp.float32).at[:N].set(feat)
    res_p = feat_p
    out_p = feat_p

    w_alls = [jnp.concatenate([W_edge[i], W_ctr[i][None]], axis=0)
              for i in range(NUM_LAYERS)]
    msg, temp0 = _msg_call(out_p, w_alls[0])
    for i in range(NUM_LAYERS):
        temp = _sc_agg(msg.reshape(NK * NP, D), temp0, gidx_pad, sidx_pad,
                       meta)
        if i < NUM_LAYERS - 1:
            out_p, msg, temp0 = _fused_call(
                temp, res_p, gamma1[i][None], beta1[i][None], W_ctr2[i],
                gamma2[i][None], beta2[i][None], w_alls[i + 1])
        else:
            out_p = _post_call(temp, res_p,
                               gamma1[i][None], beta1[i][None], W_ctr2[i],
                               gamma2[i][None], beta2[i][None])
        res_p = out_p

    return out_p[:N]


# speedup vs baseline: 1.0531x; 1.0531x over previous
import functools

import jax
import jax.numpy as jnp
from jax import lax
from jax.experimental import pallas as pl
from jax.experimental.pallas import tpu as pltpu
from jax.experimental.pallas import tpu_sc as plsc

N = 50000
D = 128
NUM_SCALES = 6
NUM_LAYERS = 4
NK = 2 * NUM_SCALES + 2

RB = 1024
RB2 = 2048
NP = 51200

EPS = 1e-5


def _dot_t(x, w):
    return lax.dot_general(x.astype(jnp.bfloat16), w.astype(jnp.bfloat16),
                           (((1,), (1,)), ((), ())),
                           preferred_element_type=jnp.float32)


def _msg_body(feat_ref, w_ref, msg_ref, temp0_ref):
    x = feat_ref[...]
    for k in range(NK):
        msg_ref[k] = _dot_t(x, w_ref[k])
    temp0_ref[...] = _dot_t(x, w_ref[NK])


def _msg_call(feat_p, w_all):
    grid = (NP // RB,)
    return pl.pallas_call(
        _msg_body,
        grid=grid,
        in_specs=[
            pl.BlockSpec((RB, D), lambda i: (i, 0)),
            pl.BlockSpec((NK + 1, D, D), lambda i: (0, 0, 0)),
        ],
        out_specs=[
            pl.BlockSpec((NK, RB, D), lambda i: (0, i, 0)),
            pl.BlockSpec((RB, D), lambda i: (i, 0)),
        ],
        out_shape=[
            jax.ShapeDtypeStruct((NK, NP, D), jnp.float32),
            jax.ShapeDtypeStruct((NP, D), jnp.float32),
        ],
    )(feat_p, w_all)


def _gn(x, gamma, beta):
    mean = jnp.mean(x, axis=-1, keepdims=True)
    var = jnp.mean((x - mean) ** 2, axis=-1, keepdims=True)
    xh = (x - mean) * lax.rsqrt(var + EPS)
    return xh * gamma + beta


def _post_body(temp_ref, res_ref, g1_ref, b1_ref, w2_ref, g2_ref, b2_ref,
               out_ref):
    t = temp_ref[...]
    x = _gn(t, g1_ref[...], b1_ref[...])
    x = jnp.maximum(x, 0.0)
    x = lax.dot_general(x, w2_ref[...], (((1,), (1,)), ((), ())),
                        preferred_element_type=jnp.float32)
    x = _gn(x, g2_ref[...], b2_ref[...])
    out_ref[...] = jnp.maximum(x + res_ref[...], 0.0)


def _post_call(temp_p, res_p, g1, b1, w2, g2, b2):
    grid = (NP // RB2,)
    return pl.pallas_call(
        _post_body,
        grid=grid,
        in_specs=[
            pl.BlockSpec((RB2, D), lambda i: (i, 0)),
            pl.BlockSpec((RB2, D), lambda i: (i, 0)),
            pl.BlockSpec((1, D), lambda i: (0, 0)),
            pl.BlockSpec((1, D), lambda i: (0, 0)),
            pl.BlockSpec((D, D), lambda i: (0, 0)),
            pl.BlockSpec((1, D), lambda i: (0, 0)),
            pl.BlockSpec((1, D), lambda i: (0, 0)),
        ],
        out_specs=pl.BlockSpec((RB2, D), lambda i: (i, 0)),
        out_shape=jax.ShapeDtypeStruct((NP, D), jnp.float32),
    )(temp_p, res_p, g1, b1, w2, g2, b2)


def _fused_body(temp_ref, res_ref, g1_ref, b1_ref, w2_ref, g2_ref, b2_ref,
                w_ref, feat_ref, msg_ref, temp0_ref):
    x = _gn(temp_ref[...], g1_ref[...], b1_ref[...])
    x = jnp.maximum(x, 0.0)
    x = _dot_t(x, w2_ref[...])
    x = _gn(x, g2_ref[...], b2_ref[...])
    f = jnp.maximum(x + res_ref[...], 0.0)
    feat_ref[...] = f
    for k in range(NK):
        msg_ref[k] = _dot_t(f, w_ref[k])
    temp0_ref[...] = _dot_t(f, w_ref[NK])


def _fused_call(temp_p, res_p, g1, b1, w2, g2, b2, w_all):
    grid = (NP // RB,)
    return pl.pallas_call(
        _fused_body,
        grid=grid,
        in_specs=[
            pl.BlockSpec((RB, D), lambda i: (i, 0)),
            pl.BlockSpec((RB, D), lambda i: (i, 0)),
            pl.BlockSpec((1, D), lambda i: (0, 0)),
            pl.BlockSpec((1, D), lambda i: (0, 0)),
            pl.BlockSpec((D, D), lambda i: (0, 0)),
            pl.BlockSpec((1, D), lambda i: (0, 0)),
            pl.BlockSpec((1, D), lambda i: (0, 0)),
            pl.BlockSpec((NK + 1, D, D), lambda i: (0, 0, 0)),
        ],
        out_specs=[
            pl.BlockSpec((RB, D), lambda i: (i, 0)),
            pl.BlockSpec((NK, RB, D), lambda i: (0, i, 0)),
            pl.BlockSpec((RB, D), lambda i: (i, 0)),
        ],
        out_shape=[
            jax.ShapeDtypeStruct((NP, D), jnp.float32),
            jax.ShapeDtypeStruct((NK, NP, D), jnp.float32),
            jax.ShapeDtypeStruct((NP, D), jnp.float32),
        ],
    )(temp_p, res_p, g1, b1, w2, g2, b2, w_all)


NCH = 6
CH = 8448
CHT = CH // 16
WIN = 128
GRP = 16 * WIN
E_TOT = 12 * 50000 + 2 * 5000
E_CAP = E_TOT + NCH * GRP

_sc_agg_cache = []


def _sc_agg(msg, temp0, gidx, sidx, meta):
    if not _sc_agg_cache:
        mesh = plsc.VectorSubcoreMesh(core_axis_name="c",
                                      subcore_axis_name="s")
        _sc_agg_cache.append(functools.partial(
            pl.kernel,
            mesh=mesh,
            out_type=jax.ShapeDtypeStruct((NP, D), jnp.float32),
            compiler_params=pltpu.CompilerParams(needs_layout_passes=False),
            scratch_types=[
                pltpu.VMEM((2, WIN), jnp.int32),
                pltpu.VMEM((2, WIN), jnp.int32),
                pltpu.VMEM((2, WIN, D), jnp.float32),
                pltpu.VMEM((2, 16), jnp.int32),
                pltpu.VMEM_SHARED((CH + 16, D), jnp.float32),
                pltpu.SemaphoreType.DMA,
                pltpu.SemaphoreType.DMA,
            ],
        )(_sc_agg_body))
    return _sc_agg_cache[0](msg, temp0, gidx, sidx, meta)


def _sc_agg_body(msg_hbm, temp0_hbm, gidx_hbm, sidx_hbm, meta_hbm, out_hbm,
                 gbuf, sbuf, rows, metav, acc, sem, sem_i):
    core = lax.axis_index("c")
    tile = lax.axis_index("s")
    pltpu.sync_copy(meta_hbm, metav)
    mv_off = metav[0]
    mv_nwt = metav[1]
    NPC = NCH // 2

    def sget(vec, j):
        return jnp.sum(jnp.where(lax.iota(jnp.int32, 16) == j, vec, 0))

    r0 = tile * CHT

    def windows(j):
        c = core * NPC + j
        off_c = sget(mv_off, c)
        nwt_c = sget(mv_nwt, c)

        def idx_start(k):
            b = k & 1
            e_off = pl.multiple_of(off_c + (k * 16 + tile) * WIN, WIN)
            pltpu.async_copy(gidx_hbm.at[pl.ds(e_off, WIN)], gbuf.at[b],
                             sem_i)
            pltpu.async_copy(sidx_hbm.at[pl.ds(e_off, WIN)], sbuf.at[b],
                             sem_i)

        def idx_wait(k):
            b = k & 1
            pltpu.make_async_copy(gidx_hbm.at[pl.ds(0, WIN)], gbuf.at[b],
                                  sem_i).wait()
            pltpu.make_async_copy(sidx_hbm.at[pl.ds(0, WIN)], sbuf.at[b],
                                  sem_i).wait()

        def gather_start(k):
            b = k & 1
            pltpu.async_copy(msg_hbm.at[gbuf.at[b]], rows.at[b], sem)

        def gather_wait(k):
            b = k & 1
            pltpu.make_async_copy(msg_hbm.at[gbuf.at[b]], rows.at[b],
                                  sem).wait()

        @pl.when(nwt_c > 0)
        def _():
            idx_start(0)

        @pl.when(nwt_c > 1)
        def _():
            idx_start(1)

        @pl.when(nwt_c > 0)
        def _():
            idx_wait(0)
            gather_start(0)

        def body(k, carry):
            @pl.when(k + 1 < nwt_c)
            def _():
                idx_wait(k + 1)
                gather_start(k + 1)

            gather_wait(k)
            b = k & 1
            pltpu.sync_copy(rows.at[b], acc.at[sbuf.at[b]], add=True)

            @pl.when(k + 2 < nwt_c)
            def _():
                idx_start(k + 2)

            return carry

        lax.fori_loop(0, nwt_c, body, 0)

    for j in range(NPC):
        base = (core * NPC + j) * CH
        pltpu.sync_copy(temp0_hbm.at[pl.ds(base + r0, CHT)],
                        acc.at[pl.ds(r0, CHT)])
        plsc.subcore_barrier()
        windows(j)
        plsc.subcore_barrier()
        pltpu.sync_copy(acc.at[pl.ds(r0, CHT)],
                        out_hbm.at[pl.ds(base + r0, CHT)])
        plsc.subcore_barrier()


def _edge_prep(u_all, gidx_all):
    cid = u_all // CH
    ind = (cid[:, None] == jnp.arange(NCH, dtype=jnp.int32)[None, :])
    ranks = jnp.cumsum(ind.astype(jnp.int32), axis=0) - 1
    cnt = jnp.sum(ind.astype(jnp.int32), axis=0)
    pc = ((cnt + GRP - 1) // GRP) * GRP
    off = jnp.concatenate([jnp.zeros((1,), jnp.int32),
                           jnp.cumsum(pc)]).astype(jnp.int32)
    rank_e = jnp.sum(jnp.where(ind, ranks, 0), axis=1)
    pos = off[cid] + rank_e
    ar = jnp.arange(E_CAP, dtype=jnp.int32)
    dummy_g = ar % 1024
    dummy_s = CH + (ar % 16)
    gidx_pad = dummy_g.at[pos].add(gidx_all - (pos % 1024))
    sidx_pad = dummy_s.at[pos].add(u_all % CH - (CH + pos % 16))
    nwt = pc // GRP
    meta = jnp.zeros((2, 16), jnp.int32).at[0, :NCH].set(off[:NCH]).at[
        1, :NCH].set(nwt)
    return gidx_pad, sidx_pad, meta


def kernel(feat, pre_u, pre_v, suc_u, suc_v, left_u, left_v, right_u,
           right_v, W_ctr, W_edge, gamma1, beta1, W_ctr2, gamma2, beta2):
    u_all = jnp.concatenate([pre_u.reshape(-1), suc_u.reshape(-1),
                             left_u, right_u]).astype(jnp.int32)
    v_all = jnp.concatenate([pre_v.reshape(-1), suc_v.reshape(-1),
                             left_v, right_v]).astype(jnp.int32)
    s_all = jnp.concatenate([
        jnp.repeat(jnp.arange(NUM_SCALES, dtype=jnp.int32), pre_u.shape[1]),
        jnp.repeat(jnp.arange(NUM_SCALES, 2 * NUM_SCALES, dtype=jnp.int32),
                   suc_u.shape[1]),
        jnp.full(left_u.shape, 2 * NUM_SCALES, jnp.int32),
        jnp.full(right_u.shape, 2 * NUM_SCALES + 1, jnp.int32),
    ])
    gidx_all = s_all * NP + v_all
    gidx_pad, sidx_pad, meta = _edge_prep(u_all, gidx_all)

    feat_p = jnp.zeros((NP, D), jnp.float32).at[:N].set(feat)
    res_p = feat_p
    out_p = feat_p

    w_alls = [jnp.concatenate([W_edge[i], W_ctr[i][None]], axis=0)
              for i in range(NUM_LAYERS)]
    msg, temp0 = _msg_call(out_p, w_alls[0])
    for i in range(NUM_LAYERS):
        temp = _sc_agg(msg.reshape(NK * NP, D), temp0, gidx_pad, sidx_pad,
                       meta)
        if i < NUM_LAYERS - 1:
            out_p, msg, temp0 = _fused_call(
                temp, res_p, gamma1[i][None], beta1[i][None], W_ctr2[i],
                gamma2[i][None], beta2[i][None], w_alls[i + 1])
        else:
            out_p = _post_call(temp, res_p,
                               gamma1[i][None], beta1[i][None], W_ctr2[i],
                               gamma2[i][None], beta2[i][None])
        res_p = out_p

    return out_p[:N]


# speedup vs baseline: 1.0630x; 1.0094x over previous
import functools

import jax
import jax.numpy as jnp
from jax import lax
from jax.experimental import pallas as pl
from jax.experimental.pallas import tpu as pltpu
from jax.experimental.pallas import tpu_sc as plsc

N = 50000
D = 128
NUM_SCALES = 6
NUM_LAYERS = 4
NK = 2 * NUM_SCALES + 2

RB = 1024
RB2 = 2048
NP = 51200

EPS = 1e-5


def _dot_t(x, w):
    return lax.dot_general(x.astype(jnp.bfloat16), w.astype(jnp.bfloat16),
                           (((1,), (1,)), ((), ())),
                           preferred_element_type=jnp.float32)


def _msg_body(feat_ref, w_ref, msg_ref, temp0_ref):
    x = feat_ref[...]
    for k in range(NK):
        msg_ref[k] = _dot_t(x, w_ref[k])
    temp0_ref[...] = _dot_t(x, w_ref[NK])


def _msg_call(feat_p, w_all):
    grid = (NP // RB,)
    return pl.pallas_call(
        _msg_body,
        grid=grid,
        in_specs=[
            pl.BlockSpec((RB, D), lambda i: (i, 0)),
            pl.BlockSpec((NK + 1, D, D), lambda i: (0, 0, 0)),
        ],
        out_specs=[
            pl.BlockSpec((NK, RB, D), lambda i: (0, i, 0)),
            pl.BlockSpec((RB, D), lambda i: (i, 0)),
        ],
        out_shape=[
            jax.ShapeDtypeStruct((NK, NP, D), jnp.float32),
            jax.ShapeDtypeStruct((NP, D), jnp.float32),
        ],
    )(feat_p, w_all)


def _gn(x, gamma, beta):
    mean = jnp.mean(x, axis=-1, keepdims=True)
    var = jnp.mean((x - mean) ** 2, axis=-1, keepdims=True)
    xh = (x - mean) * lax.rsqrt(var + EPS)
    return xh * gamma + beta


def _post_body(temp_ref, res_ref, g1_ref, b1_ref, w2_ref, g2_ref, b2_ref,
               out_ref):
    t = temp_ref[...]
    x = _gn(t, g1_ref[...], b1_ref[...])
    x = jnp.maximum(x, 0.0)
    x = lax.dot_general(x, w2_ref[...], (((1,), (1,)), ((), ())),
                        preferred_element_type=jnp.float32)
    x = _gn(x, g2_ref[...], b2_ref[...])
    out_ref[...] = jnp.maximum(x + res_ref[...], 0.0)


def _post_call(temp_p, res_p, g1, b1, w2, g2, b2):
    grid = (NP // RB2,)
    return pl.pallas_call(
        _post_body,
        grid=grid,
        in_specs=[
            pl.BlockSpec((RB2, D), lambda i: (i, 0)),
            pl.BlockSpec((RB2, D), lambda i: (i, 0)),
            pl.BlockSpec((1, D), lambda i: (0, 0)),
            pl.BlockSpec((1, D), lambda i: (0, 0)),
            pl.BlockSpec((D, D), lambda i: (0, 0)),
            pl.BlockSpec((1, D), lambda i: (0, 0)),
            pl.BlockSpec((1, D), lambda i: (0, 0)),
        ],
        out_specs=pl.BlockSpec((RB2, D), lambda i: (i, 0)),
        out_shape=jax.ShapeDtypeStruct((N, D), jnp.float32),
    )(temp_p, res_p, g1, b1, w2, g2, b2)


def _fused_body(temp_ref, res_ref, g1_ref, b1_ref, w2_ref, g2_ref, b2_ref,
                w_ref, feat_ref, msg_ref, temp0_ref):
    x = _gn(temp_ref[...], g1_ref[...], b1_ref[...])
    x = jnp.maximum(x, 0.0)
    x = _dot_t(x, w2_ref[...])
    x = _gn(x, g2_ref[...], b2_ref[...])
    f = jnp.maximum(x + res_ref[...], 0.0)
    feat_ref[...] = f
    for k in range(NK):
        msg_ref[k] = _dot_t(f, w_ref[k])
    temp0_ref[...] = _dot_t(f, w_ref[NK])


def _fused_call(temp_p, res_p, g1, b1, w2, g2, b2, w_all):
    grid = (NP // RB,)
    return pl.pallas_call(
        _fused_body,
        grid=grid,
        in_specs=[
            pl.BlockSpec((RB, D), lambda i: (i, 0)),
            pl.BlockSpec((RB, D), lambda i: (i, 0)),
            pl.BlockSpec((1, D), lambda i: (0, 0)),
            pl.BlockSpec((1, D), lambda i: (0, 0)),
            pl.BlockSpec((D, D), lambda i: (0, 0)),
            pl.BlockSpec((1, D), lambda i: (0, 0)),
            pl.BlockSpec((1, D), lambda i: (0, 0)),
            pl.BlockSpec((NK + 1, D, D), lambda i: (0, 0, 0)),
        ],
        out_specs=[
            pl.BlockSpec((RB, D), lambda i: (i, 0)),
            pl.BlockSpec((NK, RB, D), lambda i: (0, i, 0)),
            pl.BlockSpec((RB, D), lambda i: (i, 0)),
        ],
        out_shape=[
            jax.ShapeDtypeStruct((NP, D), jnp.float32),
            jax.ShapeDtypeStruct((NK, NP, D), jnp.float32),
            jax.ShapeDtypeStruct((NP, D), jnp.float32),
        ],
    )(temp_p, res_p, g1, b1, w2, g2, b2, w_all)


NCH = 6
CH = 8448
CHT = CH // 16
WIN = 128
GRP = 16 * WIN
E_TOT = 12 * 50000 + 2 * 5000
E_CAP = E_TOT + NCH * GRP

_sc_agg_cache = []


def _sc_agg(msg, temp0, gidx, sidx, meta):
    if not _sc_agg_cache:
        mesh = plsc.VectorSubcoreMesh(core_axis_name="c",
                                      subcore_axis_name="s")
        _sc_agg_cache.append(functools.partial(
            pl.kernel,
            mesh=mesh,
            out_type=jax.ShapeDtypeStruct((NP, D), jnp.float32),
            compiler_params=pltpu.CompilerParams(needs_layout_passes=False),
            scratch_types=[
                pltpu.VMEM((2, WIN), jnp.int32),
                pltpu.VMEM((2, WIN), jnp.int32),
                pltpu.VMEM((2, WIN, D), jnp.float32),
                pltpu.VMEM((2, 16), jnp.int32),
                pltpu.VMEM_SHARED((CH + 16, D), jnp.float32),
                pltpu.SemaphoreType.DMA,
                pltpu.SemaphoreType.DMA,
            ],
        )(_sc_agg_body))
    return _sc_agg_cache[0](msg, temp0, gidx, sidx, meta)


def _sc_agg_body(msg_hbm, temp0_hbm, gidx_hbm, sidx_hbm, meta_hbm, out_hbm,
                 gbuf, sbuf, rows, metav, acc, sem, sem_i):
    core = lax.axis_index("c")
    tile = lax.axis_index("s")
    pltpu.sync_copy(meta_hbm, metav)
    mv_off = metav[0]
    mv_nwt = metav[1]
    NPC = NCH // 2

    def sget(vec, j):
        return jnp.sum(jnp.where(lax.iota(jnp.int32, 16) == j, vec, 0))

    r0 = tile * CHT

    def windows(j):
        c = core * NPC + j
        off_c = sget(mv_off, c)
        nwt_c = sget(mv_nwt, c)

        def idx_start(k):
            b = k & 1
            e_off = pl.multiple_of(off_c + (k * 16 + tile) * WIN, WIN)
            pltpu.async_copy(gidx_hbm.at[pl.ds(e_off, WIN)], gbuf.at[b],
                             sem_i)
            pltpu.async_copy(sidx_hbm.at[pl.ds(e_off, WIN)], sbuf.at[b],
                             sem_i)

        def idx_wait(k):
            b = k & 1
            pltpu.make_async_copy(gidx_hbm.at[pl.ds(0, WIN)], gbuf.at[b],
                                  sem_i).wait()
            pltpu.make_async_copy(sidx_hbm.at[pl.ds(0, WIN)], sbuf.at[b],
                                  sem_i).wait()

        def gather_start(k):
            b = k & 1
            pltpu.async_copy(msg_hbm.at[gbuf.at[b]], rows.at[b], sem)

        def gather_wait(k):
            b = k & 1
            pltpu.make_async_copy(msg_hbm.at[gbuf.at[b]], rows.at[b],
                                  sem).wait()

        @pl.when(nwt_c > 0)
        def _():
            idx_start(0)

        @pl.when(nwt_c > 1)
        def _():
            idx_start(1)

        @pl.when(nwt_c > 0)
        def _():
            idx_wait(0)
            gather_start(0)

        def body(k, carry):
            @pl.when(k + 1 < nwt_c)
            def _():
                idx_wait(k + 1)
                gather_start(k + 1)

            gather_wait(k)
            b = k & 1
            pltpu.sync_copy(rows.at[b], acc.at[sbuf.at[b]], add=True)

            @pl.when(k + 2 < nwt_c)
            def _():
                idx_start(k + 2)

            return carry

        lax.fori_loop(0, nwt_c, body, 0)

    for j in range(NPC):
        base = (core * NPC + j) * CH
        pltpu.sync_copy(temp0_hbm.at[pl.ds(base + r0, CHT)],
                        acc.at[pl.ds(r0, CHT)])
        plsc.subcore_barrier()
        windows(j)
        plsc.subcore_barrier()
        pltpu.sync_copy(acc.at[pl.ds(r0, CHT)],
                        out_hbm.at[pl.ds(base + r0, CHT)])
        plsc.subcore_barrier()


def _edge_prep(u_all, gidx_all):
    cid = u_all // CH
    ind = (cid[:, None] == jnp.arange(NCH, dtype=jnp.int32)[None, :])
    ranks = jnp.cumsum(ind.astype(jnp.int32), axis=0) - 1
    cnt = jnp.sum(ind.astype(jnp.int32), axis=0)
    pc = ((cnt + GRP - 1) // GRP) * GRP
    off = jnp.concatenate([jnp.zeros((1,), jnp.int32),
                           jnp.cumsum(pc)]).astype(jnp.int32)
    rank_e = jnp.sum(jnp.where(ind, ranks, 0), axis=1)
    pos = off[cid] + rank_e
    ar = jnp.arange(E_CAP, dtype=jnp.int32)
    dummy_g = ar % 1024
    dummy_s = CH + (ar % 16)
    gidx_pad = dummy_g.at[pos].add(gidx_all - (pos % 1024))
    sidx_pad = dummy_s.at[pos].add(u_all % CH - (CH + pos % 16))
    nwt = pc // GRP
    meta = jnp.zeros((2, 16), jnp.int32).at[0, :NCH].set(off[:NCH]).at[
        1, :NCH].set(nwt)
    return gidx_pad, sidx_pad, meta


def kernel(feat, pre_u, pre_v, suc_u, suc_v, left_u, left_v, right_u,
           right_v, W_ctr, W_edge, gamma1, beta1, W_ctr2, gamma2, beta2):
    u_all = jnp.concatenate([pre_u.reshape(-1), suc_u.reshape(-1),
                             left_u, right_u]).astype(jnp.int32)
    v_all = jnp.concatenate([pre_v.reshape(-1), suc_v.reshape(-1),
                             left_v, right_v]).astype(jnp.int32)
    s_all = jnp.concatenate([
        jnp.repeat(jnp.arange(NUM_SCALES, dtype=jnp.int32), pre_u.shape[1]),
        jnp.repeat(jnp.arange(NUM_SCALES, 2 * NUM_SCALES, dtype=jnp.int32),
                   suc_u.shape[1]),
        jnp.full(left_u.shape, 2 * NUM_SCALES, jnp.int32),
        jnp.full(right_u.shape, 2 * NUM_SCALES + 1, jnp.int32),
    ])
    gidx_all = s_all * NP + v_all
    gidx_pad, sidx_pad, meta = _edge_prep(u_all, gidx_all)

    feat_p = jnp.zeros((NP, D), jnp.float32).at[:N].set(feat)
    res_p = feat_p

    w_alls = [jnp.concatenate([W_edge[i], W_ctr[i][None]], axis=0)
              for i in range(NUM_LAYERS)]
    msg, temp0 = _msg_call(feat_p, w_alls[0])
    for i in range(NUM_LAYERS):
        temp = _sc_agg(msg.reshape(NK * NP, D), temp0, gidx_pad, sidx_pad,
                       meta)
        if i < NUM_LAYERS - 1:
            res_p, msg, temp0 = _fused_call(
                temp, res_p, gamma1[i][None], beta1[i][None], W_ctr2[i],
                gamma2[i][None], beta2[i][None], w_alls[i + 1])
        else:
            return _post_call(temp, res_p,
                              gamma1[i][None], beta1[i][None], W_ctr2[i],
                              gamma2[i][None], beta2[i][None])
